# baseline (speedup 1.0000x reference)
import functools

import jax
import jax.numpy as jnp
from jax import lax
from jax.experimental import pallas as pl
from jax.experimental.pallas import tpu as pltpu
from jax.experimental.pallas import tpu_sc as plsc

N_OUT = 4096
N_IN = 4096
WT_SIZE = N_IN * N_OUT
WIN = 1 << 20
NSUB = 16
NCORE = 2
NWIN_SC = 8

PIECE = 4096
VPP = PIECE // 16
CAP = 2048
CAPR = CAP // 128
BINROWS = NWIN_SC * CAPR


def _sc_body(pieces_per_seg, nnz, r_hbm, c_hbm, w_hbm, tr_hbm, tc_hbm, tw_hbm,
             zi_hbm, zv_hbm, zrow_hbm,
             wt_hbm, bidx_hbm, bval_hbm,
             rbuf, cbuf, wbuf, idxbin, valbin, fills,
             zerobuf, spmem, semz, sems, sem_ab):
    c = lax.axis_index("c")
    s = lax.axis_index("s")
    total_pieces = sum(pieces_per_seg)
    ch = total_pieces * PIECE
    base = s * ch

    iot = lax.iota(jnp.int32, 16)
    ones = jnp.ones((16,), jnp.int32)

    pltpu.sync_copy(zrow_hbm, zerobuf)

    def fire3(pg, b):
        off = base + pg * PIECE
        is_main = off <= nnz - PIECE

        @pl.when(is_main)
        def _():
            pltpu.async_copy(r_hbm.at[pl.ds(off, PIECE)], rbuf.at[b],
                             sem_ab.at[b])
            pltpu.async_copy(c_hbm.at[pl.ds(off, PIECE)], cbuf.at[b],
                             sem_ab.at[b])
            pltpu.async_copy(w_hbm.at[pl.ds(off, PIECE)], wbuf.at[b],
                             sem_ab.at[b])

        @pl.when(jnp.logical_not(is_main))
        def _():
            pltpu.async_copy(tr_hbm, rbuf.at[b], sem_ab.at[b])
            pltpu.async_copy(tc_hbm, cbuf.at[b], sem_ab.at[b])
            pltpu.async_copy(tw_hbm, wbuf.at[b], sem_ab.at[b])

    def stage_piece(pg):
        @pl.when(pg % 2 == 0)
        def _():
            fire3(pg, 0)

        @pl.when(pg % 2 == 1)
        def _():
            fire3(pg, 1)

    def wait_piece(b):
        for _ in range(3):
            pltpu.make_async_copy(
                r_hbm.at[pl.ds(0, PIECE)], rbuf.at[b], sem_ab.at[b]).wait()

    def vbody(delta, b, j, carry):
        eidx = j * 16 + iot
        r = rbuf[b, pl.ds(j * 16, 16)]
        ci = cbuf[b, pl.ds(j * 16, 16)]
        wv = wbuf[b, pl.ds(j * 16, 16)]
        mine = (lax.shift_right_logical(ci, 11) == c) & (eidx >= delta)
        wloc = lax.shift_right_logical(ci, 8) & 7
        local = ((ci & 255) << 12) | r
        rank, _ = plsc.scan_count(wloc, mask=mine)
        fcur = plsc.load_gather(fills, [wloc])
        pos = jnp.minimum(fcur + rank - 1, CAP - 1)
        dst = wloc * CAP + pos
        hi = lax.shift_right_logical(dst, 7)
        lo = dst & 127
        plsc.store_scatter(idxbin, [hi, lo], local, mask=mine)
        plsc.store_scatter(valbin, [hi, lo], wv, mask=mine)
        plsc.addupdate_scatter(fills, [wloc], ones, mask=mine)
        return carry

    stage_piece(0)
    piece0 = 0
    for seg, npieces in enumerate(pieces_per_seg):
        pltpu.sync_copy(zi_hbm, idxbin)
        pltpu.sync_copy(zv_hbm, valbin)
        fills[...] = jnp.zeros((16,), jnp.int32)

        def piece_body(p, carry, piece0=piece0):
            pg = piece0 + p

            @pl.when(pg + 1 < total_pieces)
            def _():
                stage_piece(pg + 1)

            off = base + pg * PIECE
            delta = jnp.where(off <= nnz - PIECE, 0, off - (nnz - PIECE))

            @pl.when(pg % 2 == 0)
            def _():
                wait_piece(0)
                lax.fori_loop(0, VPP, functools.partial(vbody, delta, 0), 0)

            @pl.when(pg % 2 == 1)
            def _():
                wait_piece(1)
                lax.fori_loop(0, VPP, functools.partial(vbody, delta, 1), 0)
            return carry

        lax.fori_loop(0, npieces, piece_body, 0)
        piece0 += npieces
        pltpu.sync_copy(idxbin, bidx_hbm.at[c, s, seg])
        pltpu.sync_copy(valbin, bval_hbm.at[c, s, seg])

    nseg = len(pieces_per_seg)

    def stage_bins(w, seg, b):
        pltpu.async_copy(
            bidx_hbm.at[c, s, seg, pl.ds(w * CAPR, CAPR)],
            idxbin.at[pl.ds(b * CAPR, CAPR)], sem_ab.at[b])
        pltpu.async_copy(
            bval_hbm.at[c, s, seg, pl.ds(w * CAPR, CAPR)],
            valbin.at[pl.ds(b * CAPR, CAPR)], sem_ab.at[b])

    def wait_bins(b):
        for _ in range(2):
            pltpu.make_async_copy(
                bidx_hbm.at[c, s, 0, pl.ds(0, CAPR)],
                idxbin.at[pl.ds(b * CAPR, CAPR)], sem_ab.at[b]).wait()

    for w in range(NWIN_SC):
        for z in range(32):
            pltpu.async_copy(
                zerobuf, spmem.at[pl.ds((s * 32 + z) * 2048, 2048)], semz)
        stage_bins(w, 0, 0)
        for z in range(32):
            pltpu.make_async_copy(
                zerobuf, spmem.at[pl.ds(0, 2048)], semz).wait()
        plsc.subcore_barrier()

        for seg in range(nseg):
            b = seg % 2
            wait_bins(b)
            if seg + 1 < nseg:
                stage_bins(w, seg + 1, 1 - b)

            def fire(jj, carry, b=b):
                pltpu.async_copy(
                    valbin.at[b * CAPR + jj], spmem.at[idxbin.at[b * CAPR + jj]],
                    sems, add=True)
                return carry
            lax.fori_loop(0, CAPR, fire, 0)

            def drain(jj, carry, b=b):
                pltpu.make_async_copy(
                    valbin.at[b * CAPR], spmem.at[idxbin.at[b * CAPR]],
                    sems).wait()
                return carry
            lax.fori_loop(0, CAPR, drain, 0)
        plsc.subcore_barrier()

        rowbase = (c * NWIN_SC + w) * 256 + s * 16
        for t in range(16):
            pltpu.async_copy(spmem.at[pl.ds(s * 65536 + t * 4096, 4096)],
                             wt_hbm.at[rowbase + t], semz)
        for t in range(16):
            pltpu.make_async_copy(spmem.at[pl.ds(0, 4096)],
                                  wt_hbm.at[0], semz).wait()
        plsc.subcore_barrier()


def _build_wt(indices, weights, pieces_per_seg):
    mesh = plsc.VectorSubcoreMesh(core_axis_name="c", subcore_axis_name="s")
    nseg = len(pieces_per_seg)
    nnz = indices.shape[0]
    rs = indices[:, 0]
    cs = indices[:, 1]
    zi = jnp.zeros((BINROWS, 128), jnp.int32)
    zv = jnp.zeros((BINROWS, 128), jnp.float32)
    zrow = jnp.zeros((2048,), jnp.float32)
    wt, _, _ = pl.kernel(
        functools.partial(_sc_body, pieces_per_seg, nnz),
        out_type=(
            jax.ShapeDtypeStruct((N_IN, N_OUT), jnp.float32),
            jax.ShapeDtypeStruct((NCORE, NSUB, nseg, BINROWS, 128), jnp.int32),
            jax.ShapeDtypeStruct((NCORE, NSUB, nseg, BINROWS, 128),
                                 jnp.float32),
        ),
        mesh=mesh,
        scratch_types=[
            pltpu.VMEM((2, PIECE), jnp.int32),
            pltpu.VMEM((2, PIECE), jnp.int32),
            pltpu.VMEM((2, PIECE), jnp.float32),
            pltpu.VMEM((BINROWS, 128), jnp.int32),
            pltpu.VMEM((BINROWS, 128), jnp.float32),
            pltpu.VMEM((16,), jnp.int32),
            pltpu.VMEM((2048,), jnp.float32),
            pltpu.VMEM_SHARED((WIN,), jnp.float32),
            pltpu.SemaphoreType.DMA,
            pltpu.SemaphoreType.DMA,
            pltpu.SemaphoreType.DMA((2,)),
        ],
        compiler_params=pltpu.CompilerParams(needs_layout_passes=False),
    )(rs, cs, weights, rs[nnz - PIECE:], cs[nnz - PIECE:],
      weights[nnz - PIECE:], zi, zv, zrow)
    return wt


BN = 512
BK = 512


def _mm_body(nk, a_ref, b_ref, rest_ref, bkg_ref, o_ref, acc_ref):
    k = pl.program_id(1)

    @pl.when(k == 0)
    def _():
        acc_ref[...] = jnp.zeros_like(acc_ref)

    acc_ref[...] += jnp.dot(a_ref[...].astype(jnp.bfloat16),
                            b_ref[...].astype(jnp.bfloat16),
                            preferred_element_type=jnp.float32)

    @pl.when(k == nk - 1)
    def _():
        o_ref[...] = acc_ref[...] + rest_ref[...] * bkg_ref[...]


def _matmul_noise(inp_flat, wt, rest_col, bkg_row):
    m = inp_flat.shape[0]
    nn = wt.shape[1]
    kk = wt.shape[0]
    nk = kk // BK
    grid = (nn // BN, nk)
    return pl.pallas_call(
        functools.partial(_mm_body, nk),
        grid=grid,
        in_specs=[
            pl.BlockSpec((m, BK), lambda n, k: (0, k)),
            pl.BlockSpec((BK, BN), lambda n, k: (k, n)),
            pl.BlockSpec((m, 1), lambda n, k: (0, 0)),
            pl.BlockSpec((1, BN), lambda n, k: (0, n)),
        ],
        out_specs=pl.BlockSpec((m, BN), lambda n, k: (0, n)),
        out_shape=jax.ShapeDtypeStruct((m, nn), jnp.float32),
        scratch_shapes=[pltpu.VMEM((m, BN), jnp.float32)],
        compiler_params=pltpu.CompilerParams(
            dimension_semantics=("parallel", "arbitrary"),
        ),
    )(inp_flat, wt, rest_col, bkg_row)


def kernel(inp, indices, weights, bkg_weights):
    b, t, n_in = inp.shape
    inp_flat = jnp.reshape(inp, (b * t, n_in))

    nnz = indices.shape[0]
    pieces = -(-nnz // (NSUB * PIECE))
    nseg = max(1, -(-pieces // 6))
    lo = pieces // nseg
    nhi = pieces - nseg * lo
    pieces_per_seg = (lo + 1,) * nhi + (lo,) * (nseg - nhi)

    wt = _build_wt(indices, weights, pieces_per_seg)

    noise_key = jax.random.key(42)
    rest = jnp.sum(
        (jax.random.uniform(noise_key, (b, t, 10)) < 0.1).astype(jnp.float32),
        axis=-1)
    rest_col = jnp.reshape(rest, (b * t, 1)) / 10.0
    bkg_row = jnp.reshape(bkg_weights, (1, N_OUT))

    out_flat = _matmul_noise(inp_flat, wt, rest_col, bkg_row)
    return jnp.reshape(out_flat, (b, t, N_OUT))


# speedup vs baseline: 1.0003x; 1.0003x over previous
import functools

import jax
import jax.numpy as jnp
from jax import lax
from jax.experimental import pallas as pl
from jax.experimental.pallas import tpu as pltpu
from jax.experimental.pallas import tpu_sc as plsc

N_OUT = 4096
N_IN = 4096
WT_SIZE = N_IN * N_OUT
WIN = 1 << 20
NSUB = 16
NCORE = 2
NWIN_SC = 8

PIECE = 4096
VPP = PIECE // 16
CAP = 2048
CAPR = CAP // 128
BINROWS = NWIN_SC * CAPR


def _sc_body(pieces_per_seg, nnz, r_hbm, c_hbm, w_hbm, tr_hbm, tc_hbm, tw_hbm,
             zi_hbm, zv_hbm, zrow_hbm,
             wt_hbm, bidx_hbm, bval_hbm,
             rbuf, cbuf, wbuf, idxbin, valbin, fills,
             zerobuf, spmem, semz, sems, sem_ab):
    c = lax.axis_index("c")
    s = lax.axis_index("s")
    total_pieces = sum(pieces_per_seg)
    ch = total_pieces * PIECE
    base = s * ch

    iot = lax.iota(jnp.int32, 16)
    ones = jnp.ones((16,), jnp.int32)

    pltpu.sync_copy(zrow_hbm, zerobuf)

    def fire3(pg, b):
        off = base + pg * PIECE
        is_main = off <= nnz - PIECE

        @pl.when(is_main)
        def _():
            pltpu.async_copy(r_hbm.at[pl.ds(off, PIECE)], rbuf.at[b],
                             sem_ab.at[b])
            pltpu.async_copy(c_hbm.at[pl.ds(off, PIECE)], cbuf.at[b],
                             sem_ab.at[b])
            pltpu.async_copy(w_hbm.at[pl.ds(off, PIECE)], wbuf.at[b],
                             sem_ab.at[b])

        @pl.when(jnp.logical_not(is_main))
        def _():
            pltpu.async_copy(tr_hbm, rbuf.at[b], sem_ab.at[b])
            pltpu.async_copy(tc_hbm, cbuf.at[b], sem_ab.at[b])
            pltpu.async_copy(tw_hbm, wbuf.at[b], sem_ab.at[b])

    def stage_piece(pg):
        @pl.when(pg % 2 == 0)
        def _():
            fire3(pg, 0)

        @pl.when(pg % 2 == 1)
        def _():
            fire3(pg, 1)

    def wait_piece(b):
        for _ in range(3):
            pltpu.make_async_copy(
                r_hbm.at[pl.ds(0, PIECE)], rbuf.at[b], sem_ab.at[b]).wait()

    def vbody(delta, b, j, carry):
        eidx = j * 16 + iot
        r = rbuf[b, pl.ds(j * 16, 16)]
        ci = cbuf[b, pl.ds(j * 16, 16)]
        wv = wbuf[b, pl.ds(j * 16, 16)]
        mine = (lax.shift_right_logical(ci, 11) == c) & (eidx >= delta)
        wloc = lax.shift_right_logical(ci, 8) & 7
        local = ((ci & 255) << 12) | r
        rank, _ = plsc.scan_count(wloc, mask=mine)
        fcur = plsc.load_gather(fills, [wloc])
        pos = jnp.minimum(fcur + rank - 1, CAP - 1)
        dst = wloc * CAP + pos
        hi = lax.shift_right_logical(dst, 7)
        lo = dst & 127
        plsc.store_scatter(idxbin, [hi, lo], local, mask=mine)
        plsc.store_scatter(valbin, [hi, lo], wv, mask=mine)
        plsc.addupdate_scatter(fills, [wloc], ones, mask=mine)
        return carry

    stage_piece(0)
    piece0 = 0
    for seg, npieces in enumerate(pieces_per_seg):
        pltpu.sync_copy(zi_hbm, idxbin)
        pltpu.sync_copy(zv_hbm, valbin)
        fills[...] = jnp.zeros((16,), jnp.int32)

        def piece_body(p, carry, piece0=piece0):
            pg = piece0 + p

            @pl.when(pg + 1 < total_pieces)
            def _():
                stage_piece(pg + 1)

            off = base + pg * PIECE
            delta = jnp.where(off <= nnz - PIECE, 0, off - (nnz - PIECE))

            @pl.when(pg % 2 == 0)
            def _():
                wait_piece(0)
                lax.fori_loop(0, VPP, functools.partial(vbody, delta, 0), 0)

            @pl.when(pg % 2 == 1)
            def _():
                wait_piece(1)
                lax.fori_loop(0, VPP, functools.partial(vbody, delta, 1), 0)
            return carry

        lax.fori_loop(0, npieces, piece_body, 0)
        piece0 += npieces
        pltpu.sync_copy(idxbin, bidx_hbm.at[c, s, seg])
        pltpu.sync_copy(valbin, bval_hbm.at[c, s, seg])

    nseg = len(pieces_per_seg)

    def stage_bins(w, seg, b):
        pltpu.async_copy(
            bidx_hbm.at[c, s, seg, pl.ds(w * CAPR, CAPR)],
            idxbin.at[pl.ds(b * CAPR, CAPR)], sem_ab.at[b])
        pltpu.async_copy(
            bval_hbm.at[c, s, seg, pl.ds(w * CAPR, CAPR)],
            valbin.at[pl.ds(b * CAPR, CAPR)], sem_ab.at[b])

    def wait_bins(b):
        for _ in range(2):
            pltpu.make_async_copy(
                bidx_hbm.at[c, s, 0, pl.ds(0, CAPR)],
                idxbin.at[pl.ds(b * CAPR, CAPR)], sem_ab.at[b]).wait()

    for w in range(NWIN_SC):
        for z in range(32):
            pltpu.async_copy(
                zerobuf, spmem.at[pl.ds((s * 32 + z) * 2048, 2048)], semz)
        stage_bins(w, 0, 0)
        for z in range(32):
            pltpu.make_async_copy(
                zerobuf, spmem.at[pl.ds(0, 2048)], semz).wait()
        plsc.subcore_barrier()

        for seg in range(nseg):
            b = seg % 2
            wait_bins(b)
            if seg + 1 < nseg:
                stage_bins(w, seg + 1, 1 - b)

            def fire(jj, carry, b=b):
                pltpu.async_copy(
                    valbin.at[b * CAPR + jj], spmem.at[idxbin.at[b * CAPR + jj]],
                    sems, add=True)
                return carry
            lax.fori_loop(0, CAPR, fire, 0)

            def drain(jj, carry, b=b):
                pltpu.make_async_copy(
                    valbin.at[b * CAPR], spmem.at[idxbin.at[b * CAPR]],
                    sems).wait()
                return carry
            lax.fori_loop(0, CAPR, drain, 0)
        plsc.subcore_barrier()

        rowbase = (c * NWIN_SC + w) * 256 + s * 16
        for t in range(16):
            pltpu.async_copy(spmem.at[pl.ds(s * 65536 + t * 4096, 4096)],
                             wt_hbm.at[rowbase + t], semz)
        for t in range(16):
            pltpu.make_async_copy(spmem.at[pl.ds(0, 4096)],
                                  wt_hbm.at[0], semz).wait()
        plsc.subcore_barrier()


def _build_wt(indices, weights, pieces_per_seg):
    mesh = plsc.VectorSubcoreMesh(core_axis_name="c", subcore_axis_name="s")
    nseg = len(pieces_per_seg)
    nnz = indices.shape[0]
    rs = indices[:, 0]
    cs = indices[:, 1]
    zi = jnp.zeros((BINROWS, 128), jnp.int32)
    zv = jnp.zeros((BINROWS, 128), jnp.float32)
    zrow = jnp.zeros((2048,), jnp.float32)
    wt, _, _ = pl.kernel(
        functools.partial(_sc_body, pieces_per_seg, nnz),
        out_type=(
            jax.ShapeDtypeStruct((N_IN, N_OUT), jnp.float32),
            jax.ShapeDtypeStruct((NCORE, NSUB, nseg, BINROWS, 128), jnp.int32),
            jax.ShapeDtypeStruct((NCORE, NSUB, nseg, BINROWS, 128),
                                 jnp.float32),
        ),
        mesh=mesh,
        scratch_types=[
            pltpu.VMEM((2, PIECE), jnp.int32),
            pltpu.VMEM((2, PIECE), jnp.int32),
            pltpu.VMEM((2, PIECE), jnp.float32),
            pltpu.VMEM((BINROWS, 128), jnp.int32),
            pltpu.VMEM((BINROWS, 128), jnp.float32),
            pltpu.VMEM((16,), jnp.int32),
            pltpu.VMEM((2048,), jnp.float32),
            pltpu.VMEM_SHARED((WIN,), jnp.float32),
            pltpu.SemaphoreType.DMA,
            pltpu.SemaphoreType.DMA,
            pltpu.SemaphoreType.DMA((2,)),
        ],
        compiler_params=pltpu.CompilerParams(needs_layout_passes=False),
    )(rs, cs, weights, rs[nnz - PIECE:], cs[nnz - PIECE:],
      weights[nnz - PIECE:], zi, zv, zrow)
    return wt


BN = 512
BK = 512


def _mm_body(nk, a_ref, b_ref, rest_ref, bkg_ref, o_ref, acc_ref):
    k = pl.program_id(1)

    @pl.when(k == 0)
    def _():
        acc_ref[...] = jnp.zeros_like(acc_ref)

    acc_ref[...] += jnp.dot(a_ref[...], b_ref[...],
                            preferred_element_type=jnp.float32)

    @pl.when(k == nk - 1)
    def _():
        o_ref[...] = acc_ref[...] + rest_ref[...] * bkg_ref[...]


def _matmul_noise(inp_flat, wt, rest_col, bkg_row):
    m = inp_flat.shape[0]
    nn = wt.shape[1]
    kk = wt.shape[0]
    nk = kk // BK
    grid = (nn // BN, nk)
    return pl.pallas_call(
        functools.partial(_mm_body, nk),
        grid=grid,
        in_specs=[
            pl.BlockSpec((m, BK), lambda n, k: (0, k)),
            pl.BlockSpec((BK, BN), lambda n, k: (k, n)),
            pl.BlockSpec((m, 1), lambda n, k: (0, 0)),
            pl.BlockSpec((1, BN), lambda n, k: (0, n)),
        ],
        out_specs=pl.BlockSpec((m, BN), lambda n, k: (0, n)),
        out_shape=jax.ShapeDtypeStruct((m, nn), jnp.float32),
        scratch_shapes=[pltpu.VMEM((m, BN), jnp.float32)],
        compiler_params=pltpu.CompilerParams(
            dimension_semantics=("parallel", "arbitrary"),
        ),
    )(inp_flat, wt, rest_col, bkg_row)


def kernel(inp, indices, weights, bkg_weights):
    b, t, n_in = inp.shape
    inp_flat = jnp.reshape(inp, (b * t, n_in))

    nnz = indices.shape[0]
    pieces = -(-nnz // (NSUB * PIECE))
    nseg = max(1, -(-pieces // 6))
    lo = pieces // nseg
    nhi = pieces - nseg * lo
    pieces_per_seg = (lo + 1,) * nhi + (lo,) * (nseg - nhi)

    wt = _build_wt(indices, weights, pieces_per_seg)

    noise_key = jax.random.key(42)
    rest = jnp.sum(
        (jax.random.uniform(noise_key, (b, t, 10)) < 0.1).astype(jnp.float32),
        axis=-1)
    rest_col = jnp.reshape(rest, (b * t, 1)) / 10.0
    bkg_row = jnp.reshape(bkg_weights, (1, N_OUT))

    out_flat = _matmul_noise(inp_flat, wt, rest_col, bkg_row)
    return jnp.reshape(out_flat, (b, t, N_OUT))
